# baseline (speedup 1.0000x reference)
import functools

import jax
import jax.numpy as jnp
from jax import lax
from jax.experimental import pallas as pl
from jax.experimental.pallas import tpu as pltpu
from jax.experimental.pallas import tpu_sc as plsc

K_NN = 16
THRES = 0.12
S_GRP = 32
DB_TILE = 2048
BQ = 1024


def _prep_body(xo_ref, out_ref):
    xg = xo_ref[...]
    c = jnp.sum(xg * xg, axis=1, keepdims=True)
    c1 = c.astype(jnp.bfloat16)
    r1 = c - c1.astype(jnp.float32)
    c2 = r1.astype(jnp.bfloat16)
    c3 = (r1 - c2.astype(jnp.float32)).astype(jnp.bfloat16)
    z = jnp.zeros((xg.shape[0], 2), jnp.bfloat16)
    out_ref[...] = jnp.concatenate(
        [xg.astype(jnp.bfloat16), c1, c2, c3, z], axis=1)


def _prep(xo):
    N = xo.shape[0]
    return pl.pallas_call(
        _prep_body,
        grid=(N // DB_TILE,),
        in_specs=[pl.BlockSpec((DB_TILE, 3), lambda i: (i, 0))],
        out_specs=pl.BlockSpec((DB_TILE, 8), lambda i: (i, 0)),
        out_shape=jax.ShapeDtypeStruct((N, 8), jnp.bfloat16),
    )(xo)


def _knn_groups_body(xct_ref, xg6_ref, gid_ref, mt_ref):
    N = xg6_ref.shape[0]
    xct = xct_ref[...]
    qt = jnp.concatenate(
        [(-2.0 * xct).astype(jnp.bfloat16),
         jnp.ones((3, BQ), jnp.bfloat16),
         jnp.zeros((2, BQ), jnp.bfloat16)], axis=0)
    gpb = DB_TILE // S_GRP
    for dbt in range(N // DB_TILE):
        d = jnp.dot(xg6_ref[dbt * DB_TILE:(dbt + 1) * DB_TILE, :], qt,
                    preferred_element_type=jnp.float32)
        gm = jnp.min(d.reshape(gpb, S_GRP, BQ), axis=1)
        mt_ref[dbt * gpb:(dbt + 1) * gpb, :] = gm

    mt = mt_ref[...]
    G = mt.shape[0]
    giota = lax.broadcasted_iota(jnp.int32, (G, BQ), 0).astype(jnp.float32)
    rows = []
    for _ in range(K_NN):
        m = jnp.min(mt, axis=0, keepdims=True)
        w = jnp.min(jnp.where(mt == m, giota, 1e9), axis=0, keepdims=True)
        rows.append(w)
        mt = jnp.where(giota == w, jnp.float32(jnp.inf), mt)
    gid_ref[...] = jnp.concatenate(rows, axis=0).astype(jnp.int32)


def _knn_groups(xct, xg6):
    N = xg6.shape[0]
    Q = xct.shape[1]
    return pl.pallas_call(
        _knn_groups_body,
        grid=(Q // BQ,),
        in_specs=[
            pl.BlockSpec((3, BQ), lambda qb: (0, qb)),
            pl.BlockSpec((N, 8), lambda qb: (0, 0)),
        ],
        out_specs=pl.BlockSpec((K_NN, BQ), lambda qb: (0, qb)),
        out_shape=jax.ShapeDtypeStruct((K_NN, Q), jnp.int32),
        scratch_shapes=[pltpu.VMEM((N // S_GRP, BQ), jnp.float32)],
        compiler_params=pltpu.CompilerParams(
            dimension_semantics=("arbitrary",),
        ),
    )(xct, xg6)


def _make_sc_gather(B, D):
    info = plsc.get_sparse_core_info()
    NW = info.num_cores * info.num_subcores
    b_per_w = B // NW
    CH = min(512, b_per_w)
    n_ch = b_per_w // CH
    mesh = plsc.VectorSubcoreMesh(core_axis_name="c", subcore_axis_name="s")

    @functools.partial(
        pl.kernel, mesh=mesh,
        out_type=jax.ShapeDtypeStruct((B, D), jnp.float32),
        scratch_types=[
            pltpu.VMEM((b_per_w,), jnp.int32),
            pltpu.VMEM((CH, D), jnp.float32),
            pltpu.SemaphoreType.DMA,
        ],
    )
    def k(table_hbm, idx_hbm, out_hbm, idx_v, rows_v, sem):
        wid = lax.axis_index("s") * info.num_cores + lax.axis_index("c")
        base = wid * b_per_w
        pltpu.sync_copy(idx_hbm.at[pl.ds(base, b_per_w)], idx_v)

        def body(i, carry):
            off = pl.multiple_of(i * CH, 8)
            pltpu.async_copy(
                table_hbm.at[idx_v.at[pl.ds(off, CH)]], rows_v, sem
            ).wait()
            pltpu.sync_copy(rows_v, out_hbm.at[pl.ds(base + off, CH)])
            return carry

        lax.fori_loop(0, n_ch, body, 0)

    return k


def _rescan_mlp_body(xct_ref, gid_ref, xg_ref, w1t_ref, b1_ref, w2t_ref,
                     b2_ref, out_ref, dc_ref, ids_ref, rx_ref, ry_ref, rz_ref):
    xct = xct_ref[...]
    qx = xct[0:1]
    qy = xct[1:2]
    qz = xct[2:3]
    q_sq = (qx * qx + qy * qy) + qz * qz
    qxb = qx.astype(jnp.bfloat16).astype(jnp.float32)
    qyb = qy.astype(jnp.bfloat16).astype(jnp.float32)
    qzb = qz.astype(jnp.bfloat16).astype(jnp.float32)
    jio = lax.broadcasted_iota(jnp.int32, (S_GRP, BQ), 0).astype(jnp.float32)
    for s in range(K_NN):
        xgt = xg_ref[s].T
        px = xgt[0:S_GRP]
        py = xgt[S_GRP:2 * S_GRP]
        pz = xgt[2 * S_GRP:3 * S_GRP]
        dot = (px.astype(jnp.bfloat16).astype(jnp.float32) * qxb
               + py.astype(jnp.bfloat16).astype(jnp.float32) * qyb) \
            + pz.astype(jnp.bfloat16).astype(jnp.float32) * qzb
        cc = (px * px + py * py) + pz * pz
        sl = pl.ds(s * S_GRP, S_GRP)
        dc_ref[sl, :] = (q_sq + (-2.0) * dot) + cc
        rx_ref[sl, :] = px
        ry_ref[sl, :] = py
        rz_ref[sl, :] = pz
        gf = gid_ref[s:s + 1, :].astype(jnp.float32)
        ids_ref[sl, :] = gf * S_GRP + jio

    dc = dc_ref[...]
    ids = ids_ref[...]
    RX = rx_ref[...]
    RY = ry_ref[...]
    RZ = rz_ref[...]
    w1t = w1t_ref[...]
    b1 = b1_ref[...]
    w2t = w2t_ref[...]
    b2 = b2_ref[...]
    r0 = None
    acc = None
    for r in range(K_NN):
        m = jnp.min(dc, axis=0, keepdims=True)
        w = jnp.min(jnp.where(dc == m, ids, 1e9), axis=0, keepdims=True)
        sel = ids == w
        px = jnp.sum(jnp.where(sel, RX, 0.0), axis=0, keepdims=True)
        py = jnp.sum(jnp.where(sel, RY, 0.0), axis=0, keepdims=True)
        pz = jnp.sum(jnp.where(sel, RZ, 0.0), axis=0, keepdims=True)
        dc = jnp.where(sel, jnp.float32(jnp.inf), dc)
        resx = px - qx
        resy = py - qy
        resz = pz - qz
        if r == 0:
            r0 = (resx, resy, resz)
        ok = ((resx <= THRES) & (resx >= -THRES)
              & (resy <= THRES) & (resy >= -THRES)
              & (resz <= THRES) & (resz >= -THRES))
        resx = jnp.where(ok, resx, r0[0])
        resy = jnp.where(ok, resy, r0[1])
        resz = jnp.where(ok, resz, r0[2])
        rvec = jnp.concatenate([resx, resy, resz], axis=0)
        h1 = jnp.dot(w1t.astype(jnp.bfloat16), rvec.astype(jnp.bfloat16),
                     preferred_element_type=jnp.float32) + b1
        h1 = jnp.maximum(h1, 0.0)
        h2 = jnp.dot(w2t.astype(jnp.bfloat16), h1.astype(jnp.bfloat16),
                     preferred_element_type=jnp.float32) + b2
        acc = h2 if acc is None else jnp.maximum(acc, h2)
    out_ref[...] = acc.T


def _rescan_mlp(xct, gids, xgv, w1t, b1c, w2t, b2c):
    Q = xct.shape[1]
    D = 4 * S_GRP
    return pl.pallas_call(
        _rescan_mlp_body,
        grid=(Q // BQ,),
        in_specs=[
            pl.BlockSpec((3, BQ), lambda qb: (0, qb)),
            pl.BlockSpec((K_NN, BQ), lambda qb: (0, qb)),
            pl.BlockSpec((K_NN, BQ, D), lambda qb: (0, qb, 0)),
            pl.BlockSpec((64, 3), lambda qb: (0, 0)),
            pl.BlockSpec((64, 1), lambda qb: (0, 0)),
            pl.BlockSpec((128, 64), lambda qb: (0, 0)),
            pl.BlockSpec((128, 1), lambda qb: (0, 0)),
        ],
        out_specs=pl.BlockSpec((BQ, 128), lambda qb: (qb, 0)),
        out_shape=jax.ShapeDtypeStruct((Q, 128), jnp.float32),
        scratch_shapes=[
            pltpu.VMEM((K_NN * S_GRP, BQ), jnp.float32),
            pltpu.VMEM((K_NN * S_GRP, BQ), jnp.float32),
            pltpu.VMEM((K_NN * S_GRP, BQ), jnp.float32),
            pltpu.VMEM((K_NN * S_GRP, BQ), jnp.float32),
            pltpu.VMEM((K_NN * S_GRP, BQ), jnp.float32),
        ],
        compiler_params=pltpu.CompilerParams(
            dimension_semantics=("parallel",),
        ),
    )(xct, gids, xgv, w1t, b1c, w2t, b2c)


def kernel(x_orig, x_coarse, W1, b1, W2, b2):
    xo = x_orig[0]
    xc = x_coarse[0]
    N = xo.shape[0]
    Q = xc.shape[0]
    G = N // S_GRP

    xct = xc.T
    xg6 = _prep(xo)
    gids = _knn_groups(xct, xg6)

    xg = xo.reshape(G, S_GRP, 3)
    dbg = jnp.concatenate(
        [xg[:, :, 0], xg[:, :, 1], xg[:, :, 2],
         jnp.zeros((G, S_GRP), jnp.float32)], axis=1)
    idx_flat = gids.reshape(K_NN * Q)
    XG = _make_sc_gather(K_NN * Q, 4 * S_GRP)(dbg, idx_flat)
    xgv = XG.reshape(K_NN, Q, 4 * S_GRP)

    feat = _rescan_mlp(xct, gids, xgv, W1.T,
                       b1.reshape(64, 1), W2.T, b2.reshape(128, 1))
    return feat


# speedup vs baseline: 1.1539x; 1.1539x over previous
import functools

import jax
import jax.numpy as jnp
from jax import lax
from jax.experimental import pallas as pl
from jax.experimental.pallas import tpu as pltpu
from jax.experimental.pallas import tpu_sc as plsc

K_NN = 16
THRES = 0.12
S_GRP = 32
DB_TILE = 2048
BQ = 512


def _prep_body(xo_ref, out_ref):
    xg = xo_ref[...]
    c = jnp.sum(xg * xg, axis=1, keepdims=True)
    c1 = c.astype(jnp.bfloat16)
    r1 = c - c1.astype(jnp.float32)
    c2 = r1.astype(jnp.bfloat16)
    c3 = (r1 - c2.astype(jnp.float32)).astype(jnp.bfloat16)
    z = jnp.zeros((xg.shape[0], 2), jnp.bfloat16)
    out_ref[...] = jnp.concatenate(
        [xg.astype(jnp.bfloat16), c1, c2, c3, z], axis=1)


def _prep(xo):
    N = xo.shape[0]
    return pl.pallas_call(
        _prep_body,
        grid=(N // DB_TILE,),
        in_specs=[pl.BlockSpec((DB_TILE, 3), lambda i: (i, 0))],
        out_specs=pl.BlockSpec((DB_TILE, 8), lambda i: (i, 0)),
        out_shape=jax.ShapeDtypeStruct((N, 8), jnp.bfloat16),
    )(xo)


def _knn_groups_body(xct_ref, xg6_ref, gid_ref, mt_ref):
    N = xg6_ref.shape[0]
    xct = xct_ref[...]
    qt = jnp.concatenate(
        [(-2.0 * xct).astype(jnp.bfloat16),
         jnp.ones((3, BQ), jnp.bfloat16),
         jnp.zeros((2, BQ), jnp.bfloat16)], axis=0)
    gpb = DB_TILE // S_GRP
    for dbt in range(N // DB_TILE):
        d = jnp.dot(xg6_ref[dbt * DB_TILE:(dbt + 1) * DB_TILE, :], qt,
                    preferred_element_type=jnp.float32)
        gm = jnp.min(d.reshape(S_GRP, gpb, BQ), axis=0)
        mt_ref[dbt * gpb:(dbt + 1) * gpb, :] = gm

    mt = mt_ref[...]
    G = mt.shape[0]
    giota = lax.broadcasted_iota(jnp.int32, (G, BQ), 0).astype(jnp.float32)
    rows = []
    for _ in range(K_NN):
        m = jnp.min(mt, axis=0, keepdims=True)
        w = jnp.min(jnp.where(mt == m, giota, 1e9), axis=0, keepdims=True)
        rows.append(w)
        mt = jnp.where(giota == w, jnp.float32(jnp.inf), mt)
    gid_ref[...] = jnp.concatenate(rows, axis=0).astype(jnp.int32)


def _knn_groups(xct, xg6):
    N = xg6.shape[0]
    Q = xct.shape[1]
    return pl.pallas_call(
        _knn_groups_body,
        grid=(Q // BQ,),
        in_specs=[
            pl.BlockSpec((3, BQ), lambda qb: (0, qb)),
            pl.BlockSpec((N, 8), lambda qb: (0, 0)),
        ],
        out_specs=pl.BlockSpec((K_NN, BQ), lambda qb: (0, qb)),
        out_shape=jax.ShapeDtypeStruct((K_NN, Q), jnp.int32),
        scratch_shapes=[pltpu.VMEM((N // S_GRP, BQ), jnp.float32)],
        compiler_params=pltpu.CompilerParams(
            dimension_semantics=("arbitrary",),
        ),
    )(xct, xg6)


def _make_sc_gather(B, D):
    info = plsc.get_sparse_core_info()
    NW = info.num_cores * info.num_subcores
    b_per_w = B // NW
    CH = min(512, b_per_w)
    n_ch = b_per_w // CH
    mesh = plsc.VectorSubcoreMesh(core_axis_name="c", subcore_axis_name="s")

    @functools.partial(
        pl.kernel, mesh=mesh,
        out_type=jax.ShapeDtypeStruct((B, D), jnp.float32),
        scratch_types=[
            pltpu.VMEM((b_per_w,), jnp.int32),
            pltpu.VMEM((CH, D), jnp.float32),
            pltpu.SemaphoreType.DMA,
        ],
    )
    def k(table_hbm, idx_hbm, out_hbm, idx_v, rows_v, sem):
        wid = lax.axis_index("s") * info.num_cores + lax.axis_index("c")
        base = wid * b_per_w
        pltpu.sync_copy(idx_hbm.at[pl.ds(base, b_per_w)], idx_v)

        def body(i, carry):
            off = pl.multiple_of(i * CH, 8)
            pltpu.async_copy(
                table_hbm.at[idx_v.at[pl.ds(off, CH)]], rows_v, sem
            ).wait()
            pltpu.sync_copy(rows_v, out_hbm.at[pl.ds(base + off, CH)])
            return carry

        lax.fori_loop(0, n_ch, body, 0)

    return k


def _rescan_mlp_body(xct_ref, gid_ref, xg_ref, w1t_ref, b1_ref, w2t_ref,
                     b2_ref, out_ref, dc_ref, ids_ref, rx_ref, ry_ref, rz_ref):
    xct = xct_ref[...]
    qx = xct[0:1]
    qy = xct[1:2]
    qz = xct[2:3]
    q_sq = (qx * qx + qy * qy) + qz * qz
    qxb = qx.astype(jnp.bfloat16).astype(jnp.float32)
    qyb = qy.astype(jnp.bfloat16).astype(jnp.float32)
    qzb = qz.astype(jnp.bfloat16).astype(jnp.float32)
    jio = lax.broadcasted_iota(jnp.int32, (S_GRP, BQ), 0).astype(jnp.float32)
    for s in range(K_NN):
        xgt = xg_ref[s].T
        px = xgt[0:S_GRP]
        py = xgt[S_GRP:2 * S_GRP]
        pz = xgt[2 * S_GRP:3 * S_GRP]
        dot = (px.astype(jnp.bfloat16).astype(jnp.float32) * qxb
               + py.astype(jnp.bfloat16).astype(jnp.float32) * qyb) \
            + pz.astype(jnp.bfloat16).astype(jnp.float32) * qzb
        cc = (px * px + py * py) + pz * pz
        sl = pl.ds(s * S_GRP, S_GRP)
        dc_ref[sl, :] = (q_sq + (-2.0) * dot) + cc
        rx_ref[sl, :] = px
        ry_ref[sl, :] = py
        rz_ref[sl, :] = pz
        gi = gid_ref[s:s + 1, :]
        gpb = DB_TILE // S_GRP
        base = ((gi // gpb) * DB_TILE + (gi % gpb)).astype(jnp.float32)
        ids_ref[sl, :] = base + jio * float(gpb)

    dc = dc_ref[...]
    ids = ids_ref[...]
    RX = rx_ref[...]
    RY = ry_ref[...]
    RZ = rz_ref[...]
    w1t = w1t_ref[...]
    b1 = b1_ref[...]
    w2t = w2t_ref[...]
    b2 = b2_ref[...]
    r0 = None
    acc = None
    for r in range(K_NN):
        m = jnp.min(dc, axis=0, keepdims=True)
        w = jnp.min(jnp.where(dc == m, ids, 1e9), axis=0, keepdims=True)
        sel = ids == w
        px = jnp.sum(jnp.where(sel, RX, 0.0), axis=0, keepdims=True)
        py = jnp.sum(jnp.where(sel, RY, 0.0), axis=0, keepdims=True)
        pz = jnp.sum(jnp.where(sel, RZ, 0.0), axis=0, keepdims=True)
        dc = jnp.where(sel, jnp.float32(jnp.inf), dc)
        resx = px - qx
        resy = py - qy
        resz = pz - qz
        if r == 0:
            r0 = (resx, resy, resz)
        ok = ((resx <= THRES) & (resx >= -THRES)
              & (resy <= THRES) & (resy >= -THRES)
              & (resz <= THRES) & (resz >= -THRES))
        resx = jnp.where(ok, resx, r0[0])
        resy = jnp.where(ok, resy, r0[1])
        resz = jnp.where(ok, resz, r0[2])
        rvec = jnp.concatenate([resx, resy, resz], axis=0)
        h1 = jnp.dot(w1t.astype(jnp.bfloat16), rvec.astype(jnp.bfloat16),
                     preferred_element_type=jnp.float32) + b1
        h1 = jnp.maximum(h1, 0.0)
        h2 = jnp.dot(w2t.astype(jnp.bfloat16), h1.astype(jnp.bfloat16),
                     preferred_element_type=jnp.float32) + b2
        acc = h2 if acc is None else jnp.maximum(acc, h2)
    out_ref[...] = acc.T


def _rescan_mlp(xct, gids, xgv, w1t, b1c, w2t, b2c):
    Q = xct.shape[1]
    D = 4 * S_GRP
    return pl.pallas_call(
        _rescan_mlp_body,
        grid=(Q // BQ,),
        in_specs=[
            pl.BlockSpec((3, BQ), lambda qb: (0, qb)),
            pl.BlockSpec((K_NN, BQ), lambda qb: (0, qb)),
            pl.BlockSpec((K_NN, BQ, D), lambda qb: (0, qb, 0)),
            pl.BlockSpec((64, 3), lambda qb: (0, 0)),
            pl.BlockSpec((64, 1), lambda qb: (0, 0)),
            pl.BlockSpec((128, 64), lambda qb: (0, 0)),
            pl.BlockSpec((128, 1), lambda qb: (0, 0)),
        ],
        out_specs=pl.BlockSpec((BQ, 128), lambda qb: (qb, 0)),
        out_shape=jax.ShapeDtypeStruct((Q, 128), jnp.float32),
        scratch_shapes=[
            pltpu.VMEM((K_NN * S_GRP, BQ), jnp.float32),
            pltpu.VMEM((K_NN * S_GRP, BQ), jnp.float32),
            pltpu.VMEM((K_NN * S_GRP, BQ), jnp.float32),
            pltpu.VMEM((K_NN * S_GRP, BQ), jnp.float32),
            pltpu.VMEM((K_NN * S_GRP, BQ), jnp.float32),
        ],
        compiler_params=pltpu.CompilerParams(
            dimension_semantics=("parallel",),
        ),
    )(xct, gids, xgv, w1t, b1c, w2t, b2c)


def kernel(x_orig, x_coarse, W1, b1, W2, b2):
    xo = x_orig[0]
    xc = x_coarse[0]
    N = xo.shape[0]
    Q = xc.shape[0]
    G = N // S_GRP

    xct = xc.T
    xg6 = _prep(xo)
    gids = _knn_groups(xct, xg6)

    gpb = DB_TILE // S_GRP
    xg = xo.reshape(N // DB_TILE, S_GRP, gpb, 3).transpose(0, 2, 1, 3)
    xg = xg.reshape(G, S_GRP, 3)
    dbg = jnp.concatenate(
        [xg[:, :, 0], xg[:, :, 1], xg[:, :, 2],
         jnp.zeros((G, S_GRP), jnp.float32)], axis=1)
    idx_flat = gids.reshape(K_NN * Q)
    XG = _make_sc_gather(K_NN * Q, 4 * S_GRP)(dbg, idx_flat)
    xgv = XG.reshape(K_NN, Q, 4 * S_GRP)

    feat = _rescan_mlp(xct, gids, xgv, W1.T,
                       b1.reshape(64, 1), W2.T, b2.reshape(128, 1))
    return feat


# speedup vs baseline: 1.1541x; 1.0002x over previous
import functools

import jax
import jax.numpy as jnp
from jax import lax
from jax.experimental import pallas as pl
from jax.experimental.pallas import tpu as pltpu
from jax.experimental.pallas import tpu_sc as plsc

K_NN = 16
THRES = 0.12
S_GRP = 32
DB_TILE = 2048
BQ = 512


def _prep_body(xo_ref, out_ref):
    xg = xo_ref[...]
    c = jnp.sum(xg * xg, axis=1, keepdims=True)
    c1 = c.astype(jnp.bfloat16)
    r1 = c - c1.astype(jnp.float32)
    c2 = r1.astype(jnp.bfloat16)
    c3 = (r1 - c2.astype(jnp.float32)).astype(jnp.bfloat16)
    z = jnp.zeros((xg.shape[0], 2), jnp.bfloat16)
    out_ref[...] = jnp.concatenate(
        [xg.astype(jnp.bfloat16), c1, c2, c3, z], axis=1)


def _prep(xo):
    N = xo.shape[0]
    return pl.pallas_call(
        _prep_body,
        grid=(N // DB_TILE,),
        in_specs=[pl.BlockSpec((DB_TILE, 3), lambda i: (i, 0))],
        out_specs=pl.BlockSpec((DB_TILE, 8), lambda i: (i, 0)),
        out_shape=jax.ShapeDtypeStruct((N, 8), jnp.bfloat16),
    )(xo)


def _knn_groups_body(xct_ref, xg6_ref, gid_ref, mt_ref):
    N = xg6_ref.shape[0]
    xct = xct_ref[...]
    qt = jnp.concatenate(
        [(-2.0 * xct).astype(jnp.bfloat16),
         jnp.ones((3, BQ), jnp.bfloat16),
         jnp.zeros((2, BQ), jnp.bfloat16)], axis=0)
    gpb = DB_TILE // S_GRP
    for dbt in range(N // DB_TILE):
        d = jnp.dot(xg6_ref[dbt * DB_TILE:(dbt + 1) * DB_TILE, :], qt,
                    preferred_element_type=jnp.float32)
        gm = jnp.min(d.reshape(S_GRP, gpb, BQ), axis=0)
        mt_ref[dbt * gpb:(dbt + 1) * gpb, :] = gm

    mt = mt_ref[...]
    G = mt.shape[0]
    giota = lax.broadcasted_iota(jnp.int32, (G, BQ), 0).astype(jnp.float32)
    rows = []
    for _ in range(K_NN):
        m = jnp.min(mt, axis=0, keepdims=True)
        w = jnp.min(jnp.where(mt == m, giota, 1e9), axis=0, keepdims=True)
        rows.append(w)
        mt = jnp.where(giota == w, jnp.float32(jnp.inf), mt)
    gid_ref[...] = jnp.concatenate(rows, axis=0).astype(jnp.int32)


def _knn_groups(xct, xg6):
    N = xg6.shape[0]
    Q = xct.shape[1]
    return pl.pallas_call(
        _knn_groups_body,
        grid=(Q // BQ,),
        in_specs=[
            pl.BlockSpec((3, BQ), lambda qb: (0, qb)),
            pl.BlockSpec((N, 8), lambda qb: (0, 0)),
        ],
        out_specs=pl.BlockSpec((K_NN, BQ), lambda qb: (0, qb)),
        out_shape=jax.ShapeDtypeStruct((K_NN, Q), jnp.int32),
        scratch_shapes=[pltpu.VMEM((N // S_GRP, BQ), jnp.float32)],
        compiler_params=pltpu.CompilerParams(
            dimension_semantics=("arbitrary",),
        ),
    )(xct, xg6)


def _make_sc_gather(B, D):
    info = plsc.get_sparse_core_info()
    NW = info.num_cores * info.num_subcores
    b_per_w = B // NW
    CH = min(256, b_per_w)
    n_ch = b_per_w // CH
    assert n_ch % 2 == 0 or n_ch == 1
    mesh = plsc.VectorSubcoreMesh(core_axis_name="c", subcore_axis_name="s")

    @functools.partial(
        pl.kernel, mesh=mesh,
        out_type=jax.ShapeDtypeStruct((B, D), jnp.float32),
        scratch_types=[
            pltpu.VMEM((b_per_w,), jnp.int32),
            pltpu.VMEM((CH, D), jnp.float32),
            pltpu.VMEM((CH, D), jnp.float32),
            pltpu.SemaphoreType.DMA,
            pltpu.SemaphoreType.DMA,
        ],
    )
    def k(table_hbm, idx_hbm, out_hbm, idx_v, rows_a, rows_b, sem_a, sem_b):
        wid = lax.axis_index("s") * info.num_cores + lax.axis_index("c")
        base = wid * b_per_w
        pltpu.sync_copy(idx_hbm.at[pl.ds(base, b_per_w)], idx_v)

        def start(ch, buf, sem):
            off = pl.multiple_of(ch * CH, 8)
            pltpu.async_copy(table_hbm.at[idx_v.at[pl.ds(off, CH)]], buf, sem)

        def drain(buf, sem):
            pltpu.make_async_copy(table_hbm.at[pl.ds(0, CH)], buf, sem).wait()

        def out(ch, buf):
            off = pl.multiple_of(ch * CH, 8)
            pltpu.sync_copy(buf, out_hbm.at[pl.ds(base + off, CH)])

        start(0, rows_a, sem_a)

        def body(i, carry):
            start(2 * i + 1, rows_b, sem_b)
            drain(rows_a, sem_a)
            out(2 * i, rows_a)

            @pl.when(i + 1 < n_ch // 2)
            def _():
                start(2 * i + 2, rows_a, sem_a)

            drain(rows_b, sem_b)
            out(2 * i + 1, rows_b)
            return carry

        lax.fori_loop(0, n_ch // 2, body, 0)

    return k


def _rescan_mlp_body(xct_ref, gid_ref, xg_ref, w1t_ref, b1_ref, w2t_ref,
                     b2_ref, out_ref, dc_ref, ids_ref, rx_ref, ry_ref, rz_ref):
    xct = xct_ref[...]
    qx = xct[0:1]
    qy = xct[1:2]
    qz = xct[2:3]
    q_sq = (qx * qx + qy * qy) + qz * qz
    qxb = qx.astype(jnp.bfloat16).astype(jnp.float32)
    qyb = qy.astype(jnp.bfloat16).astype(jnp.float32)
    qzb = qz.astype(jnp.bfloat16).astype(jnp.float32)
    jio = lax.broadcasted_iota(jnp.int32, (S_GRP, BQ), 0).astype(jnp.float32)
    for s in range(K_NN):
        xgt = xg_ref[s].T
        px = xgt[0:S_GRP]
        py = xgt[S_GRP:2 * S_GRP]
        pz = xgt[2 * S_GRP:3 * S_GRP]
        dot = (px.astype(jnp.bfloat16).astype(jnp.float32) * qxb
               + py.astype(jnp.bfloat16).astype(jnp.float32) * qyb) \
            + pz.astype(jnp.bfloat16).astype(jnp.float32) * qzb
        cc = (px * px + py * py) + pz * pz
        sl = pl.ds(s * S_GRP, S_GRP)
        dc_ref[sl, :] = (q_sq + (-2.0) * dot) + cc
        rx_ref[sl, :] = px
        ry_ref[sl, :] = py
        rz_ref[sl, :] = pz
        gi = gid_ref[s:s + 1, :]
        gpb = DB_TILE // S_GRP
        base = ((gi // gpb) * DB_TILE + (gi % gpb)).astype(jnp.float32)
        ids_ref[sl, :] = base + jio * float(gpb)

    dc = dc_ref[...]
    ids = ids_ref[...]
    RX = rx_ref[...]
    RY = ry_ref[...]
    RZ = rz_ref[...]
    w1t = w1t_ref[...]
    b1 = b1_ref[...]
    w2t = w2t_ref[...]
    b2 = b2_ref[...]
    r0 = None
    acc = None
    for r in range(K_NN):
        m = jnp.min(dc, axis=0, keepdims=True)
        w = jnp.min(jnp.where(dc == m, ids, 1e9), axis=0, keepdims=True)
        sel = ids == w
        px = jnp.sum(jnp.where(sel, RX, 0.0), axis=0, keepdims=True)
        py = jnp.sum(jnp.where(sel, RY, 0.0), axis=0, keepdims=True)
        pz = jnp.sum(jnp.where(sel, RZ, 0.0), axis=0, keepdims=True)
        dc = jnp.where(sel, jnp.float32(jnp.inf), dc)
        resx = px - qx
        resy = py - qy
        resz = pz - qz
        if r == 0:
            r0 = (resx, resy, resz)
        ok = ((resx <= THRES) & (resx >= -THRES)
              & (resy <= THRES) & (resy >= -THRES)
              & (resz <= THRES) & (resz >= -THRES))
        resx = jnp.where(ok, resx, r0[0])
        resy = jnp.where(ok, resy, r0[1])
        resz = jnp.where(ok, resz, r0[2])
        rvec = jnp.concatenate([resx, resy, resz], axis=0)
        h1 = jnp.dot(w1t.astype(jnp.bfloat16), rvec.astype(jnp.bfloat16),
                     preferred_element_type=jnp.float32) + b1
        h1 = jnp.maximum(h1, 0.0)
        h2 = jnp.dot(w2t.astype(jnp.bfloat16), h1.astype(jnp.bfloat16),
                     preferred_element_type=jnp.float32) + b2
        acc = h2 if acc is None else jnp.maximum(acc, h2)
    out_ref[...] = acc.T


def _rescan_mlp(xct, gids, xgv, w1t, b1c, w2t, b2c):
    Q = xct.shape[1]
    D = 4 * S_GRP
    return pl.pallas_call(
        _rescan_mlp_body,
        grid=(Q // BQ,),
        in_specs=[
            pl.BlockSpec((3, BQ), lambda qb: (0, qb)),
            pl.BlockSpec((K_NN, BQ), lambda qb: (0, qb)),
            pl.BlockSpec((K_NN, BQ, D), lambda qb: (0, qb, 0)),
            pl.BlockSpec((64, 3), lambda qb: (0, 0)),
            pl.BlockSpec((64, 1), lambda qb: (0, 0)),
            pl.BlockSpec((128, 64), lambda qb: (0, 0)),
            pl.BlockSpec((128, 1), lambda qb: (0, 0)),
        ],
        out_specs=pl.BlockSpec((BQ, 128), lambda qb: (qb, 0)),
        out_shape=jax.ShapeDtypeStruct((Q, 128), jnp.float32),
        scratch_shapes=[
            pltpu.VMEM((K_NN * S_GRP, BQ), jnp.float32),
            pltpu.VMEM((K_NN * S_GRP, BQ), jnp.float32),
            pltpu.VMEM((K_NN * S_GRP, BQ), jnp.float32),
            pltpu.VMEM((K_NN * S_GRP, BQ), jnp.float32),
            pltpu.VMEM((K_NN * S_GRP, BQ), jnp.float32),
        ],
        compiler_params=pltpu.CompilerParams(
            dimension_semantics=("parallel",),
        ),
    )(xct, gids, xgv, w1t, b1c, w2t, b2c)


def kernel(x_orig, x_coarse, W1, b1, W2, b2):
    xo = x_orig[0]
    xc = x_coarse[0]
    N = xo.shape[0]
    Q = xc.shape[0]
    G = N // S_GRP

    xct = xc.T
    xg6 = _prep(xo)
    gids = _knn_groups(xct, xg6)

    gpb = DB_TILE // S_GRP
    xg = xo.reshape(N // DB_TILE, S_GRP, gpb, 3).transpose(0, 2, 1, 3)
    xg = xg.reshape(G, S_GRP, 3)
    dbg = jnp.concatenate(
        [xg[:, :, 0], xg[:, :, 1], xg[:, :, 2],
         jnp.zeros((G, S_GRP), jnp.float32)], axis=1)
    idx_flat = gids.reshape(K_NN * Q)
    XG = _make_sc_gather(K_NN * Q, 4 * S_GRP)(dbg, idx_flat)
    xgv = XG.reshape(K_NN, Q, 4 * S_GRP)

    feat = _rescan_mlp(xct, gids, xgv, W1.T,
                       b1.reshape(64, 1), W2.T, b2.reshape(128, 1))
    return feat
